# Initial kernel scaffold; baseline (speedup 1.0000x reference)
import jax
import jax.numpy as jnp
from jax.experimental import pallas as pl


def kernel(q, candidates, Wq, bq, Wk, bk, Wv, bv, Wo, bo, Wqh, bqh, norm_w, Wup, Wdown, k):
    raise NotImplementedError("write your pallas kernel here")


# speedup vs baseline: 1.1178x; 1.1178x over previous
import jax
import jax.numpy as jnp
from jax.experimental import pallas as pl
from jax.experimental.pallas import tpu as pltpu

B = 16
N = 4096
D = 768
H = 12
DH = 64
HID = 3072
NC = 8
CHUNK = N // NC
HC = 4
CH = HID // HC
SCALE = DH ** -0.5


def _prep_body(q_ref, z_ref, wq_ref, bq_ref, wk_ref, s_ref):
    state = q_ref[...] + z_ref[...]
    Q = jax.lax.dot_general(state, wq_ref[...],
                            (((1,), (1,)), ((), ()))) + bq_ref[...]
    for h in range(H):
        sh = jax.lax.dot_general(Q[:, h * DH:(h + 1) * DH],
                                 wk_ref[h * DH:(h + 1) * DH, :],
                                 (((1,), (0,)), ((), ())))
        s_ref[:, h, :] = sh * SCALE


def _prep(q, z, Wq, bq, Wk):
    return pl.pallas_call(
        _prep_body,
        out_shape=jax.ShapeDtypeStruct((B, H, D), jnp.float32),
    )(q, z, Wq, bq.reshape(1, D), Wk)


def _stream_body(s_ref, cand_ref, ctx_ref, rawmean_ref, m_ref, l_ref):
    c = pl.program_id(1)

    @pl.when(c == 0)
    def _():
        m_ref[...] = jnp.full((H, 1), -jnp.inf, jnp.float32)
        l_ref[...] = jnp.zeros((H, 1), jnp.float32)
        ctx_ref[0] = jnp.zeros((H, D), jnp.float32)

    s = s_ref[0]
    cand = cand_ref[0]
    rawT = jax.lax.dot_general(s, cand, (((1,), (1,)), ((), ())))
    rawmean_ref[...] = jnp.mean(rawT, axis=0, keepdims=True).reshape(
        1, 1, 1, CHUNK)

    m_old = m_ref[...]
    m_new = jnp.maximum(m_old, jnp.max(rawT, axis=1, keepdims=True))
    alpha = jnp.exp(m_old - m_new)
    w = jnp.exp(rawT - m_new)
    l_ref[...] = l_ref[...] * alpha + jnp.sum(w, axis=1, keepdims=True)
    m_ref[...] = m_new
    ctx_ref[0] = ctx_ref[0] * alpha + jax.lax.dot_general(
        w, cand, (((1,), (0,)), ((), ())))

    @pl.when(c == NC - 1)
    def _():
        ctx_ref[0] = ctx_ref[0] / l_ref[...]


def _stream(S, candidates):
    return pl.pallas_call(
        _stream_body,
        grid=(B, NC),
        in_specs=[
            pl.BlockSpec((1, H, D), lambda b, c: (b, 0, 0)),
            pl.BlockSpec((1, CHUNK, D), lambda b, c: (b, c, 0)),
        ],
        out_specs=[
            pl.BlockSpec((1, H, D), lambda b, c: (b, 0, 0)),
            pl.BlockSpec((1, 1, 1, CHUNK), lambda b, c: (b, c, 0, 0)),
        ],
        out_shape=[
            jax.ShapeDtypeStruct((B, H, D), jnp.float32),
            jax.ShapeDtypeStruct((B, NC, 1, CHUNK), jnp.float32),
        ],
        scratch_shapes=[
            pltpu.VMEM((H, 1), jnp.float32),
            pltpu.VMEM((H, 1), jnp.float32),
        ],
        compiler_params=pltpu.CompilerParams(
            dimension_semantics=("arbitrary", "arbitrary")),
    )(S, candidates)


def _selected_from_ctx(ctx, q, wv_ref, bv_ref, wo_ref, bo_ref):
    parts = []
    for h in range(H):
        parts.append(jax.lax.dot_general(
            ctx[:, h, :], wv_ref[h * DH:(h + 1) * DH, :],
            (((1,), (1,)), ((), ()))))
    out = jnp.concatenate(parts, axis=1) + bv_ref[...]
    selected = jax.lax.dot_general(out, wo_ref[...],
                                   (((1,), (1,)), ((), ()))) + bo_ref[...]
    return selected + q


def _mlp_step(h_ref, acc_s, wupg_ref, wupv_ref, wdown_ref):
    hcur = h_ref[...]
    ug = jax.lax.dot_general(hcur, wupg_ref[0], (((1,), (1,)), ((), ())))
    uv = jax.lax.dot_general(hcur, wupv_ref[0], (((1,), (1,)), ((), ())))
    sv = ug * jax.nn.sigmoid(ug) * uv
    acc_s[...] += jax.lax.dot_general(sv, wdown_ref[0],
                                      (((1,), (1,)), ((), ())))


def _layer_end(s, h_ref, acc_s, z_s, inj_s, norm_ref):
    hn = h_ref[...] + acc_s[...]
    nw = jnp.where((s % 2) == 0, norm_ref[0:1, :], norm_ref[1:2, :])
    rms = jnp.sqrt(jnp.mean(hn * hn, axis=-1, keepdims=True) + 1e-6)
    hnew = nw * (hn / rms)

    @pl.when((s % 2) == 1)
    def _():
        z_s[...] = hnew
        h_ref[...] = hnew + inj_s[...]

    @pl.when((s % 2) == 0)
    def _():
        h_ref[...] = hnew
    return hnew


def _tail_body(ctx_ref, q_ref, z_ref, wv_ref, bv_ref, wo_ref, bo_ref,
               norm_ref, wupg_ref, wupv_ref, wdown_ref, zout_ref,
               h_s, inj_s, acc_s, z_s):
    s = pl.program_id(0)
    hc = pl.program_id(1)

    @pl.when((s == 0) & (hc == 0))
    def _():
        inj = _selected_from_ctx(ctx_ref[...], q_ref[...], wv_ref, bv_ref,
                                 wo_ref, bo_ref)
        inj_s[...] = inj
        h_s[...] = z_ref[...] + inj

    @pl.when(hc == 0)
    def _():
        acc_s[...] = jnp.zeros((B, D), jnp.float32)

    _mlp_step(h_s, acc_s, wupg_ref, wupv_ref, wdown_ref)

    @pl.when(hc == HC - 1)
    def _():
        hnew = _layer_end(s, h_s, acc_s, z_s, inj_s, norm_ref)

        @pl.when(s == 3)
        def _():
            zout_ref[...] = hnew


def _tail_last_body(ctx_ref, q_ref, z_ref, rawmean_ref, wv_ref, bv_ref,
                    wo_ref, bo_ref, norm_ref, wupg_ref, wupv_ref, wdown_ref,
                    wqh_ref, bqh_ref, zout_ref, attn_ref, ti_ref, ts_ref,
                    halt_ref, h_s, inj_s, acc_s, z_s):
    s = pl.program_id(0)
    hc = pl.program_id(1)

    @pl.when((s == 0) & (hc == 0))
    def _():
        inj = _selected_from_ctx(ctx_ref[...], q_ref[...], wv_ref, bv_ref,
                                 wo_ref, bo_ref)
        inj_s[...] = inj
        h_s[...] = z_ref[...] + inj

    @pl.when(hc == 0)
    def _():
        acc_s[...] = jnp.zeros((B, D), jnp.float32)

    _mlp_step(h_s, acc_s, wupg_ref, wupv_ref, wdown_ref)

    @pl.when(hc == HC - 1)
    def _():
        hnew = _layer_end(s, h_s, acc_s, z_s, inj_s, norm_ref)

        @pl.when(s == 3)
        def _():
            zout_ref[...] = hnew
            halt_ref[...] = jnp.sum(hnew * wqh_ref[...], axis=1,
                                    keepdims=True) + bqh_ref[...]
            x = rawmean_ref[...]
            mx = jnp.max(x, axis=1, keepdims=True)
            e = jnp.exp(x - mx)
            aw = e / jnp.sum(e, axis=1, keepdims=True)
            attn_ref[...] = aw
            iota = jax.lax.broadcasted_iota(jnp.int32, (B, N), 1)
            y = aw
            ts_cols, ti_cols = [], []
            for _ in range(4):
                v = jnp.max(y, axis=1, keepdims=True)
                idx = jnp.min(jnp.where(y == v, iota, N), axis=1,
                              keepdims=True)
                ts_cols.append(v)
                ti_cols.append(idx)
                y = jnp.where(iota == idx, -1.0, y)
            ts_ref[...] = jnp.concatenate(ts_cols, axis=1)
            ti_ref[...] = jnp.concatenate(ti_cols, axis=1)


_TAIL_WSPECS = [
    pl.BlockSpec((D, D), lambda s, hc: (0, 0)),
    pl.BlockSpec((1, D), lambda s, hc: (0, 0)),
    pl.BlockSpec((D, D), lambda s, hc: (0, 0)),
    pl.BlockSpec((1, D), lambda s, hc: (0, 0)),
    pl.BlockSpec((2, D), lambda s, hc: (0, 0)),
    pl.BlockSpec((1, CH, D), lambda s, hc: (s % 2, hc, 0)),
    pl.BlockSpec((1, CH, D), lambda s, hc: (s % 2, hc + HC, 0)),
    pl.BlockSpec((1, D, CH), lambda s, hc: (s % 2, 0, hc)),
]

_TAIL_SCRATCH = [pltpu.VMEM((B, D), jnp.float32) for _ in range(4)]


def _tail(ctx, q, z, Wv, bv, Wo, bo, norm_w, Wup, Wdown):
    return pl.pallas_call(
        _tail_body,
        grid=(4, HC),
        in_specs=[
            pl.BlockSpec((B, H, D), lambda s, hc: (0, 0, 0)),
            pl.BlockSpec((B, D), lambda s, hc: (0, 0)),
            pl.BlockSpec((B, D), lambda s, hc: (0, 0)),
        ] + _TAIL_WSPECS,
        out_specs=pl.BlockSpec((B, D), lambda s, hc: (0, 0)),
        out_shape=jax.ShapeDtypeStruct((B, D), jnp.float32),
        scratch_shapes=_TAIL_SCRATCH,
        compiler_params=pltpu.CompilerParams(
            dimension_semantics=("arbitrary", "arbitrary")),
    )(ctx, q, z, Wv, bv.reshape(1, D), Wo, bo.reshape(1, D), norm_w,
      Wup, Wup, Wdown)


def _tail_last(ctx, q, z, rawmean, Wv, bv, Wo, bo, norm_w, Wup, Wdown,
               Wqh, bqh):
    return pl.pallas_call(
        _tail_last_body,
        grid=(4, HC),
        in_specs=[
            pl.BlockSpec((B, H, D), lambda s, hc: (0, 0, 0)),
            pl.BlockSpec((B, D), lambda s, hc: (0, 0)),
            pl.BlockSpec((B, D), lambda s, hc: (0, 0)),
            pl.BlockSpec((B, N), lambda s, hc: (0, 0)),
        ] + _TAIL_WSPECS + [
            pl.BlockSpec((1, D), lambda s, hc: (0, 0)),
            pl.BlockSpec((B, 1), lambda s, hc: (0, 0)),
        ],
        out_specs=[
            pl.BlockSpec((B, D), lambda s, hc: (0, 0)),
            pl.BlockSpec((B, N), lambda s, hc: (0, 0)),
            pl.BlockSpec((B, 4), lambda s, hc: (0, 0)),
            pl.BlockSpec((B, 4), lambda s, hc: (0, 0)),
            pl.BlockSpec((B, 1), lambda s, hc: (0, 0)),
        ],
        out_shape=[
            jax.ShapeDtypeStruct((B, D), jnp.float32),
            jax.ShapeDtypeStruct((B, N), jnp.float32),
            jax.ShapeDtypeStruct((B, 4), jnp.int32),
            jax.ShapeDtypeStruct((B, 4), jnp.float32),
            jax.ShapeDtypeStruct((B, 1), jnp.float32),
        ],
        scratch_shapes=_TAIL_SCRATCH,
        compiler_params=pltpu.CompilerParams(
            dimension_semantics=("arbitrary", "arbitrary")),
    )(ctx, q, z, rawmean, Wv, bv.reshape(1, D), Wo, bo.reshape(1, D),
      norm_w, Wup, Wup, Wdown, Wqh,
      jnp.broadcast_to(bqh.reshape(1, 1), (B, 1)))


def kernel(q, candidates, Wq, bq, Wk, bk, Wv, bv, Wo, bo, Wqh, bqh,
           norm_w, Wup, Wdown, k):
    z = q
    for r in range(3):
        S = _prep(q, z, Wq, bq, Wk)
        ctx, rawmean = _stream(S, candidates)
        if r < 2:
            z = _tail(ctx, q, z, Wv, bv, Wo, bo, norm_w, Wup, Wdown)
        else:
            rm = rawmean.reshape(B, N)
            z, attn_w, ti, ts, halt = _tail_last(
                ctx, q, z, rm, Wv, bv, Wo, bo, norm_w, Wup, Wdown, Wqh, bqh)
    return (z, attn_w, ti, ts, halt)
